# baseline (speedup 1.0000x reference)
import jax
import jax.numpy as jnp
from jax.experimental import pallas as pl
from jax.experimental.pallas import tpu as pltpu

D_MODEL = 2048
D_FF = 8192
N_BLK = 32
D1 = D_FF // N_BLK
N_TOK = 4096

T_TOK = 1024
T_X = 16
C = T_X * N_BLK
T_XP = 8
CP = T_XP * N_BLK
TX_PREP = N_TOK // (D_FF // CP)


def _prep_kernel(w1_ref, w2_ref, x_ref, w1b_ref, w2b_ref, xb_ref):
    w1b_ref[...] = w1_ref[...].reshape(D_MODEL, CP).astype(jnp.bfloat16)
    w2b_ref[...] = (
        jnp.transpose(w2_ref[...], (1, 0, 2))
        .reshape(CP, D_MODEL)
        .astype(jnp.bfloat16)
    )
    xb_ref[...] = x_ref[...].astype(jnp.bfloat16)


def _ff_kernel(qm_ref, x_ref, w1_ref, w2_ref, b2_ref, oh_ref, res_ref):
    j = pl.program_id(1)

    mid = jnp.dot(x_ref[...], w1_ref[...], preferred_element_type=jnp.float32)

    xi = jax.lax.broadcasted_iota(jnp.int32, (D1, C), 0)
    ci = jax.lax.broadcasted_iota(jnp.int32, (D1, C), 1)
    expand = (xi == j * T_X + ci // N_BLK).astype(jnp.bfloat16)
    qm_f = qm_ref[...].astype(jnp.bfloat16)
    qm_exp = jnp.dot(qm_f, expand, preferred_element_type=jnp.float32)

    yi = jax.lax.broadcasted_iota(jnp.int32, (T_TOK, C), 1) % N_BLK
    sel = qm_exp == yi.astype(jnp.float32)

    oh_ref[...] = sel.astype(jnp.float32).reshape(T_TOK, T_X, N_BLK)

    zero = jnp.zeros((), jnp.bfloat16)
    relu = jnp.where(sel, jnp.maximum(mid, 0.0).astype(jnp.bfloat16), zero)

    part = jnp.dot(relu, w2_ref[...], preferred_element_type=jnp.float32)

    @pl.when(j == 0)
    def _():
        res_ref[...] = b2_ref[...] + part

    @pl.when(j != 0)
    def _():
        res_ref[...] = res_ref[...] + part


def kernel(quant_mask, mask, x, w1, w2, b2):
    del mask
    b2_2d = b2.reshape(1, D_MODEL)

    w1_b, w2_b, x_b = pl.pallas_call(
        _prep_kernel,
        grid=(D_FF // CP,),
        in_specs=[
            pl.BlockSpec((D_MODEL, T_XP, N_BLK), lambda j: (0, j, 0)),
            pl.BlockSpec((N_BLK, T_XP, D_MODEL), lambda j: (0, j, 0)),
            pl.BlockSpec((TX_PREP, D_MODEL), lambda j: (j, 0)),
        ],
        out_specs=[
            pl.BlockSpec((D_MODEL, CP), lambda j: (0, j)),
            pl.BlockSpec((CP, D_MODEL), lambda j: (j, 0)),
            pl.BlockSpec((TX_PREP, D_MODEL), lambda j: (j, 0)),
        ],
        out_shape=[
            jax.ShapeDtypeStruct((D_MODEL, D_FF), jnp.bfloat16),
            jax.ShapeDtypeStruct((D_FF, D_MODEL), jnp.bfloat16),
            jax.ShapeDtypeStruct((N_TOK, D_MODEL), jnp.bfloat16),
        ],
    )(w1, w2, x)

    grid = (N_TOK // T_TOK, D_FF // C)
    oh, res = pl.pallas_call(
        _ff_kernel,
        grid=grid,
        in_specs=[
            pl.BlockSpec((T_TOK, D1), lambda i, j: (i, 0)),
            pl.BlockSpec((T_TOK, D_MODEL), lambda i, j: (i, 0)),
            pl.BlockSpec((D_MODEL, C), lambda i, j: (0, j)),
            pl.BlockSpec((C, D_MODEL), lambda i, j: (j, 0)),
            pl.BlockSpec((1, D_MODEL), lambda i, j: (0, 0)),
        ],
        out_specs=[
            pl.BlockSpec((T_TOK, T_X, N_BLK), lambda i, j: (i, j, 0)),
            pl.BlockSpec((T_TOK, D_MODEL), lambda i, j: (i, 0)),
        ],
        out_shape=[
            jax.ShapeDtypeStruct((N_TOK, D1, N_BLK), jnp.float32),
            jax.ShapeDtypeStruct((N_TOK, D_MODEL), jnp.float32),
        ],
        compiler_params=pltpu.CompilerParams(
            dimension_semantics=("parallel", "arbitrary"),
        ),
    )(quant_mask, x_b, w1_b, w2_b, b2_2d)

    return (oh, res)


# speedup vs baseline: 1.1775x; 1.1775x over previous
import jax
import jax.numpy as jnp
from jax.experimental import pallas as pl
from jax.experimental.pallas import tpu as pltpu

D_MODEL = 2048
D_FF = 8192
N_BLK = 32
D1 = D_FF // N_BLK
N_TOK = 4096

T_TOK = 1024
T_X = 16
C = T_X * N_BLK

T_XP = 8
CP = T_XP * N_BLK
TP_PREP = N_TOK // (D_FF // CP)

T_OH = 512


def _oh_kernel(qm_ref, oh_ref):
    xi = jax.lax.broadcasted_iota(jnp.int32, (D1, D_FF), 0)
    ci = jax.lax.broadcasted_iota(jnp.int32, (D1, D_FF), 1)
    expand = (xi == ci // N_BLK).astype(jnp.bfloat16)
    qm_f = qm_ref[...].astype(jnp.bfloat16)
    qm_exp = jnp.dot(qm_f, expand, preferred_element_type=jnp.float32)
    yi = jax.lax.broadcasted_iota(jnp.int32, (T_OH, D_FF), 1) % N_BLK
    oh_ref[...] = (qm_exp == yi.astype(jnp.float32)).astype(jnp.float32)


def _prep_kernel(w1_ref, w2_ref, x_ref, w1b_ref, w2b_ref, xb_ref):
    w1b_ref[...] = w1_ref[...].reshape(D_MODEL, CP).astype(jnp.bfloat16)
    w2b_ref[...] = (
        jnp.transpose(w2_ref[...], (1, 0, 2))
        .reshape(CP, D_MODEL)
        .astype(jnp.bfloat16)
    )
    xb_ref[...] = x_ref[...].astype(jnp.bfloat16)


def _ff_kernel(oh_ref, x_ref, w1_ref, w2_ref, b2_ref, res_ref):
    j = pl.program_id(1)

    mid = jnp.dot(x_ref[...], w1_ref[...], preferred_element_type=jnp.float32)

    zero = jnp.zeros((), jnp.bfloat16)
    relu = jnp.where(
        oh_ref[...] != 0.0, jnp.maximum(mid, 0.0).astype(jnp.bfloat16), zero
    )

    part = jnp.dot(relu, w2_ref[...], preferred_element_type=jnp.float32)

    @pl.when(j == 0)
    def _():
        res_ref[...] = b2_ref[...] + part

    @pl.when(j != 0)
    def _():
        res_ref[...] = res_ref[...] + part


def kernel(quant_mask, mask, x, w1, w2, b2):
    del mask
    b2_2d = b2.reshape(1, D_MODEL)

    oh_flat = pl.pallas_call(
        _oh_kernel,
        grid=(N_TOK // T_OH,),
        in_specs=[pl.BlockSpec((T_OH, D1), lambda i: (i, 0))],
        out_specs=pl.BlockSpec((T_OH, D_FF), lambda i: (i, 0)),
        out_shape=jax.ShapeDtypeStruct((N_TOK, D_FF), jnp.float32),
    )(quant_mask)

    w1_b, w2_b, x_b = pl.pallas_call(
        _prep_kernel,
        grid=(D_FF // CP,),
        in_specs=[
            pl.BlockSpec((D_MODEL, T_XP, N_BLK), lambda j: (0, j, 0)),
            pl.BlockSpec((N_BLK, T_XP, D_MODEL), lambda j: (0, j, 0)),
            pl.BlockSpec((TP_PREP, D_MODEL), lambda j: (j, 0)),
        ],
        out_specs=[
            pl.BlockSpec((D_MODEL, CP), lambda j: (0, j)),
            pl.BlockSpec((CP, D_MODEL), lambda j: (j, 0)),
            pl.BlockSpec((TP_PREP, D_MODEL), lambda j: (j, 0)),
        ],
        out_shape=[
            jax.ShapeDtypeStruct((D_MODEL, D_FF), jnp.bfloat16),
            jax.ShapeDtypeStruct((D_FF, D_MODEL), jnp.bfloat16),
            jax.ShapeDtypeStruct((N_TOK, D_MODEL), jnp.bfloat16),
        ],
    )(w1, w2, x)

    grid = (N_TOK // T_TOK, D_FF // C)
    res = pl.pallas_call(
        _ff_kernel,
        grid=grid,
        in_specs=[
            pl.BlockSpec((T_TOK, C), lambda i, j: (i, j)),
            pl.BlockSpec((T_TOK, D_MODEL), lambda i, j: (i, 0)),
            pl.BlockSpec((D_MODEL, C), lambda i, j: (0, j)),
            pl.BlockSpec((C, D_MODEL), lambda i, j: (j, 0)),
            pl.BlockSpec((1, D_MODEL), lambda i, j: (0, 0)),
        ],
        out_specs=pl.BlockSpec((T_TOK, D_MODEL), lambda i, j: (i, 0)),
        out_shape=jax.ShapeDtypeStruct((N_TOK, D_MODEL), jnp.float32),
        compiler_params=pltpu.CompilerParams(
            dimension_semantics=("parallel", "arbitrary"),
        ),
    )(oh_flat, x_b, w1_b, w2_b, b2_2d)

    return (oh_flat.reshape(N_TOK, D1, N_BLK), res)


# speedup vs baseline: 2.0669x; 1.7554x over previous
import jax
import jax.numpy as jnp
from jax.experimental import pallas as pl
from jax.experimental.pallas import tpu as pltpu

D_MODEL = 2048
D_FF = 8192
N_BLK = 32
D1 = D_FF // N_BLK
N_TOK = 4096

T_TOK = 1024
T_Y = 2
C = T_Y * D1

T_OH = 256
T_PREP = 512


def _oh_kernel(qm_ref, oh_ref):
    q3 = qm_ref[...].reshape(T_OH, 1, D1)
    yi = jax.lax.broadcasted_iota(jnp.int32, (T_OH, N_BLK, D1), 1)
    oh_ref[...] = (yi == q3).astype(jnp.float32)


def _prep_kernel(w2_ref, x_ref, w2b_ref, xb_ref):
    w2b_ref[...] = w2_ref[...].astype(jnp.bfloat16)
    xb_ref[...] = x_ref[...].astype(jnp.bfloat16)


def _ff_kernel(qm_ref, x_ref, w1_ref, w2_ref, b2_ref, res_ref):
    j = pl.program_id(1)

    mid = jnp.dot(x_ref[...], w1_ref[...].astype(jnp.bfloat16),
                  preferred_element_type=jnp.float32)

    q = qm_ref[...]
    zero = jnp.zeros((), jnp.bfloat16)
    parts = []
    for yy in range(T_Y):
        m = mid[:, yy * D1:(yy + 1) * D1]
        sel = q == (j * T_Y + yy)
        parts.append(jnp.where(sel, jnp.maximum(m, 0.0).astype(jnp.bfloat16),
                               zero))
    relu = jnp.concatenate(parts, axis=1)

    part = jnp.dot(relu, w2_ref[...], preferred_element_type=jnp.float32)

    @pl.when(j == 0)
    def _():
        res_ref[...] = b2_ref[...] + part

    @pl.when(j != 0)
    def _():
        res_ref[...] = res_ref[...] + part


def kernel(quant_mask, mask, x, w1, w2, b2):
    del mask
    b2_2d = b2.reshape(1, D_MODEL)
    w1_ym = jnp.transpose(w1, (0, 2, 1)).reshape(D_MODEL, D_FF)
    w2_ym = w2.reshape(D_FF, D_MODEL)

    oh_ym = pl.pallas_call(
        _oh_kernel,
        grid=(N_TOK // T_OH,),
        in_specs=[pl.BlockSpec((T_OH, D1), lambda i: (i, 0))],
        out_specs=pl.BlockSpec((T_OH, N_BLK, D1), lambda i: (i, 0, 0)),
        out_shape=jax.ShapeDtypeStruct((N_TOK, N_BLK, D1), jnp.float32),
    )(quant_mask)

    w2_b, x_b = pl.pallas_call(
        _prep_kernel,
        grid=(D_FF // T_PREP,),
        in_specs=[
            pl.BlockSpec((T_PREP, D_MODEL), lambda j: (j, 0)),
            pl.BlockSpec((N_TOK // (D_FF // T_PREP), D_MODEL),
                         lambda j: (j, 0)),
        ],
        out_specs=[
            pl.BlockSpec((T_PREP, D_MODEL), lambda j: (j, 0)),
            pl.BlockSpec((N_TOK // (D_FF // T_PREP), D_MODEL),
                         lambda j: (j, 0)),
        ],
        out_shape=[
            jax.ShapeDtypeStruct((D_FF, D_MODEL), jnp.bfloat16),
            jax.ShapeDtypeStruct((N_TOK, D_MODEL), jnp.bfloat16),
        ],
    )(w2_ym, x)

    grid = (N_TOK // T_TOK, D_FF // C)
    res = pl.pallas_call(
        _ff_kernel,
        grid=grid,
        in_specs=[
            pl.BlockSpec((T_TOK, D1), lambda i, j: (i, 0)),
            pl.BlockSpec((T_TOK, D_MODEL), lambda i, j: (i, 0)),
            pl.BlockSpec((D_MODEL, C), lambda i, j: (0, j)),
            pl.BlockSpec((C, D_MODEL), lambda i, j: (j, 0)),
            pl.BlockSpec((1, D_MODEL), lambda i, j: (0, 0)),
        ],
        out_specs=pl.BlockSpec((T_TOK, D_MODEL), lambda i, j: (i, 0)),
        out_shape=jax.ShapeDtypeStruct((N_TOK, D_MODEL), jnp.float32),
        compiler_params=pltpu.CompilerParams(
            dimension_semantics=("parallel", "arbitrary"),
        ),
    )(quant_mask, x_b, w1_ym, w2_b, b2_2d)

    return (jnp.transpose(oh_ym, (0, 2, 1)), res)


# speedup vs baseline: 2.2164x; 1.0723x over previous
import jax
import jax.numpy as jnp
from jax.experimental import pallas as pl
from jax.experimental.pallas import tpu as pltpu

D_MODEL = 2048
D_FF = 8192
N_BLK = 32
D1 = D_FF // N_BLK
N_TOK = 4096

T_TOK = 1024
T_Y = 2
C = T_Y * D1

T_OH = 256
T_PREP = 512


def _oh_kernel(qm_ref, oh_ref):
    q3 = qm_ref[...].reshape(T_OH, 1, D1)
    yi = jax.lax.broadcasted_iota(jnp.int32, (T_OH, N_BLK, D1), 1)
    oh_ref[...] = (yi == q3).astype(jnp.float32)


def _prep_kernel(w2_ref, x_ref, w2b_ref, xb_ref):
    w2b_ref[...] = w2_ref[...].astype(jnp.bfloat16)
    xb_ref[...] = x_ref[...].astype(jnp.bfloat16)


def _w1_kernel(w1_ref, w1b_ref):
    w1b_ref[...] = w1_ref[...].reshape(D_MODEL, 8 * D1).astype(jnp.bfloat16)


def _ff_kernel(qm_ref, x_ref, w1_ref, w2_ref, b2_ref, res_ref):
    j = pl.program_id(1)

    mid = jnp.dot(x_ref[...], w1_ref[...], preferred_element_type=jnp.float32)

    q = qm_ref[...]
    zero = jnp.zeros((), jnp.bfloat16)
    parts = []
    for yy in range(T_Y):
        m = mid[:, yy * D1:(yy + 1) * D1]
        sel = q == (j * T_Y + yy)
        parts.append(jnp.where(sel, jnp.maximum(m, 0.0).astype(jnp.bfloat16),
                               zero))
    relu = jnp.concatenate(parts, axis=1)

    part = jnp.dot(relu, w2_ref[...], preferred_element_type=jnp.float32)

    @pl.when(j == 0)
    def _():
        res_ref[...] = b2_ref[...] + part

    @pl.when(j != 0)
    def _():
        res_ref[...] = res_ref[...] + part


def kernel(quant_mask, mask, x, w1, w2, b2):
    del mask
    b2_2d = b2.reshape(1, D_MODEL)
    w1_t = jnp.transpose(w1, (0, 2, 1))
    w2_ym = w2.reshape(D_FF, D_MODEL)

    w1_b = pl.pallas_call(
        _w1_kernel,
        grid=(N_BLK // 8,),
        in_specs=[pl.BlockSpec((D_MODEL, 8, D1), lambda j: (0, j, 0))],
        out_specs=pl.BlockSpec((D_MODEL, 8 * D1), lambda j: (0, j)),
        out_shape=jax.ShapeDtypeStruct((D_MODEL, D_FF), jnp.bfloat16),
    )(w1_t)

    oh_ym = pl.pallas_call(
        _oh_kernel,
        grid=(N_TOK // T_OH,),
        in_specs=[pl.BlockSpec((T_OH, D1), lambda i: (i, 0))],
        out_specs=pl.BlockSpec((T_OH, N_BLK, D1), lambda i: (i, 0, 0)),
        out_shape=jax.ShapeDtypeStruct((N_TOK, N_BLK, D1), jnp.float32),
    )(quant_mask)

    w2_b, x_b = pl.pallas_call(
        _prep_kernel,
        grid=(D_FF // T_PREP,),
        in_specs=[
            pl.BlockSpec((T_PREP, D_MODEL), lambda j: (j, 0)),
            pl.BlockSpec((N_TOK // (D_FF // T_PREP), D_MODEL),
                         lambda j: (j, 0)),
        ],
        out_specs=[
            pl.BlockSpec((T_PREP, D_MODEL), lambda j: (j, 0)),
            pl.BlockSpec((N_TOK // (D_FF // T_PREP), D_MODEL),
                         lambda j: (j, 0)),
        ],
        out_shape=[
            jax.ShapeDtypeStruct((D_FF, D_MODEL), jnp.bfloat16),
            jax.ShapeDtypeStruct((N_TOK, D_MODEL), jnp.bfloat16),
        ],
    )(w2_ym, x)

    grid = (N_TOK // T_TOK, D_FF // C)
    res = pl.pallas_call(
        _ff_kernel,
        grid=grid,
        in_specs=[
            pl.BlockSpec((T_TOK, D1), lambda i, j: (i, 0)),
            pl.BlockSpec((T_TOK, D_MODEL), lambda i, j: (i, 0)),
            pl.BlockSpec((D_MODEL, C), lambda i, j: (0, j)),
            pl.BlockSpec((C, D_MODEL), lambda i, j: (j, 0)),
            pl.BlockSpec((1, D_MODEL), lambda i, j: (0, 0)),
        ],
        out_specs=pl.BlockSpec((T_TOK, D_MODEL), lambda i, j: (i, 0)),
        out_shape=jax.ShapeDtypeStruct((N_TOK, D_MODEL), jnp.float32),
        compiler_params=pltpu.CompilerParams(
            dimension_semantics=("parallel", "arbitrary"),
        ),
    )(quant_mask, x_b, w1_b, w2_b, b2_2d)

    return (jnp.transpose(oh_ym, (0, 2, 1)), res)
